# baseline (speedup 1.0000x reference)
import functools

import jax
import jax.numpy as jnp
from jax import lax
from jax.experimental import pallas as pl
from jax.experimental.pallas import tpu as pltpu
from jax.experimental.pallas import tpu_sc as plsc

N = 10000
D = 128
E = 320000

NC = 2
NS = 16
NW = NC * NS

NPAD = 10240
RPT = NPAD // NS

EPT = 10240
EPAD = EPT * NW
NB = EPT // 128

_MESH = plsc.VectorSubcoreMesh(core_axis_name="c", subcore_axis_name="s")


@functools.partial(
    pl.kernel,
    out_type=jax.ShapeDtypeStruct((NC, NPAD), jnp.float32),
    mesh=_MESH,
    scratch_types=[
        pltpu.VMEM((NB, 128), jnp.int32),
        pltpu.VMEM((128,), jnp.float32),
        pltpu.VMEM_SHARED((NPAD,), jnp.float32),
    ],
)
def _deg_kernel(dst_hbm, ones_hbm, zeros1_hbm, deg_hbm, dst_v, ones_v, sdeg):
    c = lax.axis_index("c")
    s = lax.axis_index("s")
    w = s * NC + c
    pltpu.sync_copy(zeros1_hbm.at[pl.ds(s * RPT, RPT)], sdeg.at[pl.ds(s * RPT, RPT)])
    pltpu.sync_copy(dst_hbm.at[pl.ds(w * NB, NB)], dst_v)
    pltpu.sync_copy(ones_hbm, ones_v)
    plsc.subcore_barrier()

    def body(j, carry):
        pltpu.sync_copy(ones_v, sdeg.at[dst_v.at[j]], add=True)
        return carry

    lax.fori_loop(0, NB, body, 0)
    plsc.subcore_barrier()
    pltpu.sync_copy(sdeg.at[pl.ds(s * RPT, RPT)], deg_hbm.at[c, pl.ds(s * RPT, RPT)])


@functools.partial(
    pl.kernel,
    out_type=jax.ShapeDtypeStruct((NC, NPAD, D), jnp.float32),
    mesh=_MESH,
    scratch_types=[
        pltpu.VMEM((NB, 128), jnp.int32),
        [pltpu.VMEM((128,), jnp.int32) for _ in range(2)],
        [pltpu.VMEM((128,), jnp.int32) for _ in range(2)],
        [pltpu.VMEM((128, D), jnp.float32) for _ in range(2)],
        pltpu.VMEM_SHARED((NPAD, D), jnp.float32),
        [pltpu.SemaphoreType.DMA for _ in range(2)],
        [pltpu.SemaphoreType.DMA for _ in range(2)],
    ],
)
def _edge_kernel(y_hbm, pk_hbm, zeros2_hbm, part_hbm,
                 pk_v, srcs, dsts, rows, acc, gsem, ssem):
    c = lax.axis_index("c")
    s = lax.axis_index("s")
    w = s * NC + c
    pltpu.sync_copy(zeros2_hbm, acc.at[pl.ds(s * RPT, RPT)])
    pltpu.sync_copy(pk_hbm.at[pl.ds(w * NB, NB)], pk_v)
    plsc.subcore_barrier()

    def unpack(j, q):
        for k in range(8):
            v = pk_v[j, pl.ds(k * 16, 16)]
            srcs[q][pl.ds(k * 16, 16)] = lax.bitwise_and(v, 0xFFFF)
            dsts[q][pl.ds(k * 16, 16)] = lax.shift_right_logical(v, 16)

    def gather(q):
        pltpu.async_copy(y_hbm.at[srcs[q]], rows[q], gsem[q])

    def gwait(q):
        pltpu.make_async_copy(y_hbm.at[srcs[q]], rows[q], gsem[q]).wait()

    def scatter(q):
        pltpu.async_copy(rows[q], acc.at[dsts[q]], ssem[q], add=True)

    def sdrain(q):
        pltpu.make_async_copy(rows[q], acc.at[dsts[q]], ssem[q]).wait()

    unpack(jnp.int32(0), 0)
    gather(0)

    def slot(j, par, other, first):
        gwait(par)
        scatter(par)

        @pl.when(j + 1 < NB)
        def _():
            if first:
                @pl.when(j >= 1)
                def _():
                    sdrain(other)
            else:
                sdrain(other)
            unpack(j + 1, other)
            gather(other)

    def body(i, carry):
        slot(2 * i, 0, 1, True)
        slot(2 * i + 1, 1, 0, False)
        return carry

    lax.fori_loop(0, NB // 2, body, 0)
    sdrain(0)
    sdrain(1)
    plsc.subcore_barrier()
    pltpu.sync_copy(acc.at[pl.ds(s * RPT, RPT)],
                    part_hbm.at[c, pl.ds(s * RPT, RPT)])


def _mm_body(x_ref, w_ref, xw_ref):
    xw_ref[...] = jnp.dot(x_ref[...], w_ref[...],
                          preferred_element_type=jnp.float32)


def _matmul(x_pad, W):
    blk = 1024
    return pl.pallas_call(
        _mm_body,
        grid=(NPAD // blk,),
        in_specs=[
            pl.BlockSpec((blk, D), lambda i: (i, 0)),
            pl.BlockSpec((D, D), lambda i: (0, 0)),
        ],
        out_specs=pl.BlockSpec((blk, D), lambda i: (i, 0)),
        out_shape=jax.ShapeDtypeStruct((NPAD, D), jnp.float32),
    )(x_pad, W)


def _scale_body(xw_ref, degp_ref, y_ref, dis_ref):
    deg = degp_ref[0, :] + degp_ref[1, :] + 1.0
    dis = lax.rsqrt(deg)
    y_ref[...] = xw_ref[...] * dis[:, None]
    dis_ref[...] = dis


def _scale(xw, degp):
    blk = 1024
    return pl.pallas_call(
        _scale_body,
        grid=(NPAD // blk,),
        in_specs=[
            pl.BlockSpec((blk, D), lambda i: (i, 0)),
            pl.BlockSpec((NC, blk), lambda i: (0, i)),
        ],
        out_specs=[
            pl.BlockSpec((blk, D), lambda i: (i, 0)),
            pl.BlockSpec((blk,), lambda i: (i,)),
        ],
        out_shape=[
            jax.ShapeDtypeStruct((NPAD, D), jnp.float32),
            jax.ShapeDtypeStruct((NPAD,), jnp.float32),
        ],
    )(xw, degp)


def _comb_body(part_ref, y_ref, dis_ref, b_ref, out_ref):
    tot = part_ref[0] + part_ref[1] + y_ref[...]
    out = tot * dis_ref[...][:, None] + b_ref[...][None, :]
    out_ref[...] = jnp.maximum(out, 0.0)


def _combine(part, y, dis, b):
    blk = 1024
    return pl.pallas_call(
        _comb_body,
        grid=(NPAD // blk,),
        in_specs=[
            pl.BlockSpec((NC, blk, D), lambda i: (0, i, 0)),
            pl.BlockSpec((blk, D), lambda i: (i, 0)),
            pl.BlockSpec((blk,), lambda i: (i,)),
            pl.BlockSpec((D,), lambda i: (0,)),
        ],
        out_specs=pl.BlockSpec((blk, D), lambda i: (i, 0)),
        out_shape=jax.ShapeDtypeStruct((NPAD, D), jnp.float32),
    )(part, y, dis, b)


def kernel(x, edge_index, W, b):
    src = edge_index[0].astype(jnp.int32)
    dst = edge_index[1].astype(jnp.int32)
    pad = N + jnp.arange(EPAD - E, dtype=jnp.int32) % (NPAD - N)
    srcp = jnp.concatenate([src, pad]).reshape(EPAD // 128, 128)
    dstp = jnp.concatenate([dst, pad]).reshape(EPAD // 128, 128)
    packed = jnp.bitwise_or(jnp.left_shift(dstp, 16), srcp)

    x_pad = jnp.pad(x, ((0, NPAD - N), (0, 0)))

    ones2 = jnp.ones((128,), jnp.float32)
    zeros1 = jnp.zeros((NPAD,), jnp.float32)
    zeros2 = jnp.zeros((RPT, D), jnp.float32)

    degp = _deg_kernel(dstp, ones2, zeros1)
    xw = _matmul(x_pad, W)
    y, dis = _scale(xw, degp)
    part = _edge_kernel(y, packed, zeros2)
    out = _combine(part, y, dis, b)
    return out[:N]


# speedup vs baseline: 1.1399x; 1.1399x over previous
import functools

import jax
import jax.numpy as jnp
from jax import lax
from jax.experimental import pallas as pl
from jax.experimental.pallas import tpu as pltpu
from jax.experimental.pallas import tpu_sc as plsc

N = 10000
D = 128
E = 320000

NC = 2
NS = 16
NW = NC * NS

NPAD = 10240
RPT = NPAD // NS

EPT = 10240
EPAD = EPT * NW
NB = EPT // 128

_MESH = plsc.VectorSubcoreMesh(core_axis_name="c", subcore_axis_name="s")


@functools.partial(
    pl.kernel,
    out_type=jax.ShapeDtypeStruct((NC, NPAD), jnp.float32),
    mesh=_MESH,
    scratch_types=[
        pltpu.VMEM((NB, 128), jnp.int32),
        pltpu.VMEM((128,), jnp.float32),
        pltpu.VMEM_SHARED((NPAD,), jnp.float32),
    ],
)
def _deg_kernel(dst_hbm, ones_hbm, zeros1_hbm, deg_hbm, dst_v, ones_v, sdeg):
    c = lax.axis_index("c")
    s = lax.axis_index("s")
    w = s * NC + c
    pltpu.sync_copy(zeros1_hbm.at[pl.ds(s * RPT, RPT)], sdeg.at[pl.ds(s * RPT, RPT)])
    pltpu.sync_copy(dst_hbm.at[pl.ds(w * NB, NB)], dst_v)
    pltpu.sync_copy(ones_hbm, ones_v)
    plsc.subcore_barrier()

    def body(j, carry):
        pltpu.sync_copy(ones_v, sdeg.at[dst_v.at[j]], add=True)
        return carry

    lax.fori_loop(0, NB, body, 0)
    plsc.subcore_barrier()
    pltpu.sync_copy(sdeg.at[pl.ds(s * RPT, RPT)], deg_hbm.at[c, pl.ds(s * RPT, RPT)])


@functools.partial(
    pl.kernel,
    out_type=jax.ShapeDtypeStruct((NC, NPAD, D), jnp.float32),
    mesh=_MESH,
    scratch_types=[
        pltpu.VMEM((NB, 128), jnp.int32),
        [pltpu.VMEM((128,), jnp.int32) for _ in range(2)],
        [pltpu.VMEM((128,), jnp.int32) for _ in range(2)],
        [pltpu.VMEM((128, D), jnp.float32) for _ in range(2)],
        pltpu.VMEM_SHARED((NPAD, D), jnp.float32),
        [pltpu.SemaphoreType.DMA for _ in range(2)],
        [pltpu.SemaphoreType.DMA for _ in range(2)],
    ],
)
def _edge_kernel(y_hbm, pk_hbm, zeros2_hbm, part_hbm,
                 pk_v, srcs, dsts, rows, acc, gsem, ssem):
    c = lax.axis_index("c")
    s = lax.axis_index("s")
    w = s * NC + c
    pltpu.sync_copy(zeros2_hbm, acc.at[pl.ds(s * RPT, RPT)])
    pltpu.sync_copy(pk_hbm.at[pl.ds(w * NB, NB)], pk_v)
    plsc.subcore_barrier()

    def unpack(j, q):
        for k in range(8):
            v = pk_v[j, pl.ds(k * 16, 16)]
            srcs[q][pl.ds(k * 16, 16)] = lax.bitwise_and(v, 0xFFFF)
            dsts[q][pl.ds(k * 16, 16)] = lax.shift_right_logical(v, 16)

    def gather(q):
        pltpu.async_copy(y_hbm.at[srcs[q]], rows[q], gsem[q])

    def gwait(q):
        pltpu.make_async_copy(y_hbm.at[srcs[q]], rows[q], gsem[q]).wait()

    def scatter(q):
        pltpu.sync_copy(rows[q], acc.at[dsts[q]], add=True)

    unpack(jnp.int32(0), 0)
    gather(0)

    def body(i, carry):
        j0 = 2 * i
        unpack(j0 + 1, 1)
        gather(1)
        gwait(0)
        scatter(0)

        @pl.when(i < NB // 2 - 1)
        def _():
            unpack(j0 + 2, 0)
            gather(0)

        gwait(1)
        scatter(1)
        return carry

    lax.fori_loop(0, NB // 2, body, 0)
    plsc.subcore_barrier()
    pltpu.sync_copy(acc.at[pl.ds(s * RPT, RPT)],
                    part_hbm.at[c, pl.ds(s * RPT, RPT)])


def _mm_body(x_ref, w_ref, xw_ref):
    xw_ref[...] = jnp.dot(x_ref[...], w_ref[...],
                          preferred_element_type=jnp.float32)


def _matmul(x_pad, W):
    blk = 1024
    return pl.pallas_call(
        _mm_body,
        grid=(NPAD // blk,),
        in_specs=[
            pl.BlockSpec((blk, D), lambda i: (i, 0)),
            pl.BlockSpec((D, D), lambda i: (0, 0)),
        ],
        out_specs=pl.BlockSpec((blk, D), lambda i: (i, 0)),
        out_shape=jax.ShapeDtypeStruct((NPAD, D), jnp.float32),
    )(x_pad, W)


def _scale_body(xw_ref, degp_ref, y_ref, dis_ref):
    deg = degp_ref[0, :] + degp_ref[1, :] + 1.0
    dis = lax.rsqrt(deg)
    y_ref[...] = xw_ref[...] * dis[:, None]
    dis_ref[...] = dis


def _scale(xw, degp):
    blk = 1024
    return pl.pallas_call(
        _scale_body,
        grid=(NPAD // blk,),
        in_specs=[
            pl.BlockSpec((blk, D), lambda i: (i, 0)),
            pl.BlockSpec((NC, blk), lambda i: (0, i)),
        ],
        out_specs=[
            pl.BlockSpec((blk, D), lambda i: (i, 0)),
            pl.BlockSpec((blk,), lambda i: (i,)),
        ],
        out_shape=[
            jax.ShapeDtypeStruct((NPAD, D), jnp.float32),
            jax.ShapeDtypeStruct((NPAD,), jnp.float32),
        ],
    )(xw, degp)


def _comb_body(part_ref, y_ref, dis_ref, b_ref, out_ref):
    tot = part_ref[0] + part_ref[1] + y_ref[...]
    out = tot * dis_ref[...][:, None] + b_ref[...][None, :]
    out_ref[...] = jnp.maximum(out, 0.0)


def _combine(part, y, dis, b):
    blk = 1024
    return pl.pallas_call(
        _comb_body,
        grid=(NPAD // blk,),
        in_specs=[
            pl.BlockSpec((NC, blk, D), lambda i: (0, i, 0)),
            pl.BlockSpec((blk, D), lambda i: (i, 0)),
            pl.BlockSpec((blk,), lambda i: (i,)),
            pl.BlockSpec((D,), lambda i: (0,)),
        ],
        out_specs=pl.BlockSpec((blk, D), lambda i: (i, 0)),
        out_shape=jax.ShapeDtypeStruct((NPAD, D), jnp.float32),
    )(part, y, dis, b)


def kernel(x, edge_index, W, b):
    src = edge_index[0].astype(jnp.int32)
    dst = edge_index[1].astype(jnp.int32)
    pad = N + jnp.arange(EPAD - E, dtype=jnp.int32) % (NPAD - N)
    srcp = jnp.concatenate([src, pad]).reshape(EPAD // 128, 128)
    dstp = jnp.concatenate([dst, pad]).reshape(EPAD // 128, 128)
    packed = jnp.bitwise_or(jnp.left_shift(dstp, 16), srcp)

    x_pad = jnp.pad(x, ((0, NPAD - N), (0, 0)))

    ones2 = jnp.ones((128,), jnp.float32)
    zeros1 = jnp.zeros((NPAD,), jnp.float32)
    zeros2 = jnp.zeros((RPT, D), jnp.float32)

    degp = _deg_kernel(dstp, ones2, zeros1)
    xw = _matmul(x_pad, W)
    y, dis = _scale(xw, degp)
    part = _edge_kernel(y, packed, zeros2)
    out = _combine(part, y, dis, b)
    return out[:N]
